# baseline (speedup 1.0000x reference)
import functools

import jax
import jax.numpy as jnp
from jax import lax
from jax.experimental import pallas as pl
from jax.experimental.pallas import tpu as pltpu
from jax.experimental.pallas import tpu_sc as plsc

_NC = 2
_NS = 16
_CHUNK = 128
_G = 24
_D = 8


def _mesh():
    return plsc.VectorSubcoreMesh(core_axis_name="c", subcore_axis_name="s")


_SC_PARAMS = pltpu.CompilerParams(use_tc_tiling_on_sc=False)


def _deg_pass(n_acc, cpw, n_grps):
    stripe = n_acc // _NS

    @functools.partial(
        pl.kernel,
        out_type=jax.ShapeDtypeStruct((_NC, n_acc, _D), jnp.float32),
        mesh=_mesh(),
        compiler_params=_SC_PARAMS,
        scratch_types=[
            pltpu.VMEM_SHARED((n_acc, _D), jnp.float32),
            pltpu.VMEM((_CHUNK, _D), jnp.float32),
            pltpu.VMEM((_G, _CHUNK), jnp.int32),
            pltpu.VMEM((_G, _CHUNK), jnp.int32),
            pltpu.SemaphoreType.DMA,
            pltpu.SemaphoreType.DMA,
        ],
    )
    def k(dst2d, zeros_hbm, ones_hbm, out, acc_sh, onesv, db0, db1, ss0, ss1):
        dbs, sss = (db0, db1), (ss0, ss1)
        c = lax.axis_index("c")
        s = lax.axis_index("s")
        r0 = s * stripe
        pltpu.sync_copy(zeros_hbm.at[pl.ds(r0, stripe)],
                        acc_sh.at[pl.ds(r0, stripe)])
        pltpu.sync_copy(ones_hbm, onesv)
        plsc.subcore_barrier()
        e0 = (c * _NS + s) * cpw
        dummy = zeros_hbm.at[pl.ds(0, _CHUNK)]

        def grp2(i, carry):
            for p in range(2):
                dbuf, ssem = dbs[p], sss[p]

                @pl.when(i > 0)
                def _():
                    for g in range(_G):
                        pltpu.make_async_copy(dummy, onesv, ssem).wait()

                base = e0 + (2 * i + p) * _G
                pltpu.sync_copy(dst2d.at[pl.ds(base, _G)], dbuf)
                for g in range(_G):
                    pltpu.async_copy(onesv, acc_sh.at[dbuf.at[g]], ssem,
                                     add=True)
            return carry

        lax.fori_loop(0, n_grps // 2, grp2, 0)
        for p in range(2):
            for g in range(_G):
                pltpu.make_async_copy(dummy, onesv, sss[p]).wait()
        plsc.subcore_barrier()
        pltpu.sync_copy(acc_sh.at[pl.ds(r0, stripe)],
                        out.at[c, pl.ds(r0, stripe)])

    return k


def _agg_pass(n_acc, cpw, n_grps):
    stripe = n_acc // _NS

    @functools.partial(
        pl.kernel,
        out_type=jax.ShapeDtypeStruct((_NC, n_acc, _D), jnp.float32),
        mesh=_mesh(),
        compiler_params=_SC_PARAMS,
        scratch_types=[
            pltpu.VMEM_SHARED((n_acc, _D), jnp.float32),
            pltpu.VMEM((_G, _CHUNK, _D), jnp.float32),
            pltpu.VMEM((_G, _CHUNK, _D), jnp.float32),
            pltpu.VMEM((_G, _CHUNK), jnp.int32),
            pltpu.VMEM((_G, _CHUNK), jnp.int32),
            pltpu.VMEM((_G, _CHUNK), jnp.int32),
            pltpu.VMEM((_G, _CHUNK), jnp.int32),
            pltpu.SemaphoreType.DMA,
            pltpu.SemaphoreType.DMA,
            pltpu.SemaphoreType.DMA,
        ],
    )
    def k(src2d, dst2d, table, zeros_hbm, out, acc_sh,
          rows0, rows1, sb0, sb1, db0, db1, gsem, ss0, ss1):
        rws, sbs, dbs, sss = (rows0, rows1), (sb0, sb1), (db0, db1), (ss0, ss1)
        c = lax.axis_index("c")
        s = lax.axis_index("s")
        r0 = s * stripe
        pltpu.sync_copy(zeros_hbm.at[pl.ds(r0, stripe)],
                        acc_sh.at[pl.ds(r0, stripe)])
        plsc.subcore_barrier()
        e0 = (c * _NS + s) * cpw
        dummy = zeros_hbm.at[pl.ds(0, _CHUNK)]

        def grp2(i, carry):
            for p in range(2):
                rows, sbuf, dbuf, ssem = rws[p], sbs[p], dbs[p], sss[p]

                @pl.when(i > 0)
                def _():
                    for g in range(_G):
                        pltpu.make_async_copy(dummy, rows.at[g], ssem).wait()

                base = e0 + (2 * i + p) * _G
                pltpu.sync_copy(src2d.at[pl.ds(base, _G)], sbuf)
                pltpu.sync_copy(dst2d.at[pl.ds(base, _G)], dbuf)
                descs = [pltpu.async_copy(table.at[sbuf.at[g]], rows.at[g],
                                          gsem) for g in range(_G)]
                for d in descs:
                    d.wait()
                for g in range(_G):
                    pltpu.async_copy(rows.at[g], acc_sh.at[dbuf.at[g]], ssem,
                                     add=True)
            return carry

        lax.fori_loop(0, n_grps // 2, grp2, 0)
        for p in range(2):
            for g in range(_G):
                pltpu.make_async_copy(dummy, rws[p].at[g], sss[p]).wait()
        plsc.subcore_barrier()
        pltpu.sync_copy(acc_sh.at[pl.ds(r0, stripe)],
                        out.at[c, pl.ds(r0, stripe)])

    return k


def _tc1_body(degp_ref, x_ref, dinv_ref, u1_ref):
    deg = degp_ref[0, :, 0:1] + degp_ref[1, :, 0:1] + 1.0
    dinv = lax.rsqrt(deg)
    dinv_ref[...] = dinv
    u1_ref[...] = x_ref[...] * dinv


def _tc2_body(accp_ref, u1_ref, dinv_ref, w1_ref, b1_ref, w2_ref, u2_ref):
    dinv = dinv_ref[...]
    a1 = dinv * (accp_ref[0] + accp_ref[1] + u1_ref[...])
    h = jnp.maximum(
        jnp.dot(a1, w1_ref[...], preferred_element_type=jnp.float32)
        + b1_ref[...], 0.0)
    p = jnp.dot(h, w2_ref[...], preferred_element_type=jnp.float32)
    u2_ref[...] = p * dinv


def _tc3_body(accp_ref, u2_ref, dinv_ref, b2_ref, out_ref):
    out_ref[...] = (dinv_ref[...] * (accp_ref[0] + accp_ref[1] + u2_ref[...])
                    + b2_ref[...])


def _row_specs(rblk, shapes):
    specs = []
    for shp in shapes:
        if len(shp) == 3:
            specs.append(pl.BlockSpec((_NC, rblk, shp[2]),
                                      lambda i: (0, i, 0)))
        elif shp[0] == "full":
            specs.append(pl.BlockSpec(shp[1], lambda i, s=shp[1]: (0,) * len(s)))
        else:
            specs.append(pl.BlockSpec((rblk, shp[1]), lambda i: (i, 0)))
    return specs


def kernel(x, edge_index, W1, b1, W2, b2):
    n = x.shape[0]
    e = edge_index.shape[1]
    f_in = x.shape[1]
    f_hid = W1.shape[1]
    f_out = W2.shape[1]

    n_acc = ((n + 1 + _NS * 8 - 1) // (_NS * 8)) * (_NS * 8)
    while n_acc % 128 != 0:
        n_acc += _NS * 8

    per = _NC * _NS * 2 * _G * _CHUNK
    n_chunks = ((e + per - 1) // per) * per // _CHUNK
    cpw = n_chunks // (_NC * _NS)
    n_grps = cpw // _G
    ep = n_chunks * _CHUNK

    pad = jnp.full((ep - e,), n, dtype=jnp.int32)
    src2d = jnp.concatenate([edge_index[0], pad]).reshape(n_chunks, _CHUNK)
    dst2d = jnp.concatenate([edge_index[1], pad]).reshape(n_chunks, _CHUNK)

    xp = jnp.zeros((n_acc, _D), jnp.float32).at[:n, :f_in].set(x)
    w1p = jnp.zeros((_D, f_hid), jnp.float32).at[:f_in].set(W1)
    w2p = jnp.zeros((f_hid, _D), jnp.float32).at[:, :f_out].set(W2)
    b2p = jnp.zeros((1, _D), jnp.float32).at[0, :f_out].set(b2)
    zeros8 = jnp.zeros((n_acc, _D), jnp.float32)
    ones = jnp.ones((_CHUNK, _D), jnp.float32)

    degp = _deg_pass(n_acc, cpw, n_grps)(dst2d, zeros8, ones)

    rblk = n_acc // 32 if n_acc % 32 == 0 else n_acc
    while n_acc % rblk or rblk % 8:
        rblk //= 2
    grid = n_acc // rblk
    dinv, u1 = pl.pallas_call(
        _tc1_body,
        grid=(grid,),
        in_specs=_row_specs(rblk, [(_NC, n_acc, _D), (None, _D)]),
        out_specs=_row_specs(rblk, [(None, 1), (None, _D)]),
        out_shape=[jax.ShapeDtypeStruct((n_acc, 1), jnp.float32),
                   jax.ShapeDtypeStruct((n_acc, _D), jnp.float32)],
    )(degp, xp)

    acc1 = _agg_pass(n_acc, cpw, n_grps)(src2d, dst2d, u1, zeros8)

    u2 = pl.pallas_call(
        _tc2_body,
        grid=(grid,),
        in_specs=_row_specs(rblk, [
            (_NC, n_acc, _D), (None, _D), (None, 1),
            ("full", (_D, f_hid)), ("full", (1, f_hid)),
            ("full", (f_hid, _D))]),
        out_specs=_row_specs(rblk, [(None, _D)]),
        out_shape=[jax.ShapeDtypeStruct((n_acc, _D), jnp.float32)],
    )(acc1, u1, dinv, w1p, b1.reshape(1, f_hid), w2p)[0]

    acc2 = _agg_pass(n_acc, cpw, n_grps)(src2d, dst2d, u2, zeros8)

    out = pl.pallas_call(
        _tc3_body,
        grid=(grid,),
        in_specs=_row_specs(rblk, [
            (_NC, n_acc, _D), (None, _D), (None, 1),
            ("full", (1, _D))]),
        out_specs=_row_specs(rblk, [(None, _D)]),
        out_shape=[jax.ShapeDtypeStruct((n_acc, _D), jnp.float32)],
    )(acc2, u2, dinv, b2p)[0]

    return out[:n, :f_out]


# speedup vs baseline: 1.3066x; 1.3066x over previous
import functools

import jax
import jax.numpy as jnp
from jax import lax
from jax.experimental import pallas as pl
from jax.experimental.pallas import tpu as pltpu
from jax.experimental.pallas import tpu_sc as plsc

_NC = 2
_NS = 16
_CHUNK = 128
_G = 16
_D = 8


def _mesh():
    return plsc.VectorSubcoreMesh(core_axis_name="c", subcore_axis_name="s")


_SC_PARAMS = pltpu.CompilerParams(use_tc_tiling_on_sc=False)


def _deg_pass(n_acc, cpw, n_grps):
    stripe = n_acc // _NS

    @functools.partial(
        pl.kernel,
        out_type=jax.ShapeDtypeStruct((_NC, n_acc, _D), jnp.float32),
        mesh=_mesh(),
        compiler_params=_SC_PARAMS,
        scratch_types=[
            pltpu.VMEM_SHARED((n_acc, _D), jnp.float32),
            pltpu.VMEM((_CHUNK, _D), jnp.float32),
            pltpu.VMEM((_G, _CHUNK), jnp.int32),
            pltpu.VMEM((_G, _CHUNK), jnp.int32),
            pltpu.SemaphoreType.DMA,
            pltpu.SemaphoreType.DMA,
        ],
    )
    def k(dst2d, zeros_hbm, ones_hbm, out, acc_sh, onesv, db0, db1, ss0, ss1):
        dbs, sss = (db0, db1), (ss0, ss1)
        c = lax.axis_index("c")
        s = lax.axis_index("s")
        r0 = s * stripe
        pltpu.sync_copy(zeros_hbm.at[pl.ds(r0, stripe)],
                        acc_sh.at[pl.ds(r0, stripe)])
        pltpu.sync_copy(ones_hbm, onesv)
        plsc.subcore_barrier()
        e0 = (c * _NS + s) * cpw
        dummy = zeros_hbm.at[pl.ds(0, _CHUNK)]

        def grp2(i, carry):
            for p in range(2):
                dbuf, ssem = dbs[p], sss[p]

                @pl.when(i > 0)
                def _():
                    for g in range(_G):
                        pltpu.make_async_copy(dummy, onesv, ssem).wait()

                base = e0 + (2 * i + p) * _G
                pltpu.sync_copy(dst2d.at[pl.ds(base, _G)], dbuf)
                for g in range(_G):
                    pltpu.async_copy(onesv, acc_sh.at[dbuf.at[g]], ssem,
                                     add=True)
            return carry

        lax.fori_loop(0, n_grps // 2, grp2, 0)
        for p in range(2):
            for g in range(_G):
                pltpu.make_async_copy(dummy, onesv, sss[p]).wait()
        plsc.subcore_barrier()
        pltpu.sync_copy(acc_sh.at[pl.ds(r0, stripe)],
                        out.at[c, pl.ds(r0, stripe)])

    return k


def _agg_pass(n_acc, cpw, n_grps):
    stripe = n_acc // _NS

    @functools.partial(
        pl.kernel,
        out_type=jax.ShapeDtypeStruct((_NC, n_acc, _D), jnp.float32),
        mesh=_mesh(),
        compiler_params=_SC_PARAMS,
        scratch_types=[
            pltpu.VMEM_SHARED((n_acc, _D), jnp.float32),
            pltpu.VMEM((_G, _CHUNK, _D), jnp.float32),
            pltpu.VMEM((_G, _CHUNK, _D), jnp.float32),
            pltpu.VMEM((_G, _CHUNK), jnp.int32),
            pltpu.VMEM((_G, _CHUNK), jnp.int32),
            pltpu.VMEM((_G, _CHUNK), jnp.int32),
            pltpu.VMEM((_G, _CHUNK), jnp.int32),
            pltpu.SemaphoreType.DMA,
            pltpu.SemaphoreType.DMA,
            pltpu.SemaphoreType.DMA,
        ],
    )
    def k(src2d, dst2d, table, zeros_hbm, out, acc_sh,
          rows0, rows1, sb0, sb1, db0, db1, gsem, ss0, ss1):
        rws, sbs, dbs, sss = (rows0, rows1), (sb0, sb1), (db0, db1), (ss0, ss1)
        c = lax.axis_index("c")
        s = lax.axis_index("s")
        r0 = s * stripe
        pltpu.sync_copy(zeros_hbm.at[pl.ds(r0, stripe)],
                        acc_sh.at[pl.ds(r0, stripe)])
        plsc.subcore_barrier()
        e0 = (c * _NS + s) * cpw
        dummy = zeros_hbm.at[pl.ds(0, _CHUNK)]

        def grp2(i, carry):
            for p in range(2):
                rows, sbuf, dbuf, ssem = rws[p], sbs[p], dbs[p], sss[p]

                @pl.when(i > 0)
                def _():
                    for g in range(_G):
                        pltpu.make_async_copy(dummy, rows.at[g], ssem).wait()

                base = e0 + (2 * i + p) * _G
                pltpu.sync_copy(src2d.at[pl.ds(base, _G)], sbuf)
                pltpu.sync_copy(dst2d.at[pl.ds(base, _G)], dbuf)
                descs = [pltpu.async_copy(table.at[sbuf.at[g]], rows.at[g],
                                          gsem) for g in range(_G)]
                for d in descs:
                    d.wait()
                for g in range(_G):
                    pltpu.async_copy(rows.at[g], acc_sh.at[dbuf.at[g]], ssem,
                                     add=True)
            return carry

        lax.fori_loop(0, n_grps // 2, grp2, 0)
        for p in range(2):
            for g in range(_G):
                pltpu.make_async_copy(dummy, rws[p].at[g], sss[p]).wait()
        plsc.subcore_barrier()
        pltpu.sync_copy(acc_sh.at[pl.ds(r0, stripe)],
                        out.at[c, pl.ds(r0, stripe)])

    return k


def _tc1_body(degp_ref, x_ref, dinv_ref, u1_ref):
    deg = degp_ref[0, :, 0:1] + degp_ref[1, :, 0:1] + 1.0
    dinv = lax.rsqrt(deg)
    dinv_ref[...] = dinv
    u1_ref[...] = x_ref[...] * dinv


def _tc2_body(accp_ref, u1_ref, dinv_ref, w1_ref, b1_ref, w2_ref, u2_ref):
    dinv = dinv_ref[...]
    a1 = dinv * (accp_ref[0] + accp_ref[1] + u1_ref[...])
    h = jnp.maximum(
        jnp.dot(a1, w1_ref[...], preferred_element_type=jnp.float32)
        + b1_ref[...], 0.0)
    p = jnp.dot(h, w2_ref[...], preferred_element_type=jnp.float32)
    u2_ref[...] = p * dinv


def _tc3_body(accp_ref, u2_ref, dinv_ref, b2_ref, out_ref):
    out_ref[...] = (dinv_ref[...] * (accp_ref[0] + accp_ref[1] + u2_ref[...])
                    + b2_ref[...])


def _row_specs(rblk, shapes):
    specs = []
    for shp in shapes:
        if len(shp) == 3:
            specs.append(pl.BlockSpec((_NC, rblk, shp[2]),
                                      lambda i: (0, i, 0)))
        elif shp[0] == "full":
            specs.append(pl.BlockSpec(shp[1], lambda i, s=shp[1]: (0,) * len(s)))
        else:
            specs.append(pl.BlockSpec((rblk, shp[1]), lambda i: (i, 0)))
    return specs


def kernel(x, edge_index, W1, b1, W2, b2):
    n = x.shape[0]
    e = edge_index.shape[1]
    f_in = x.shape[1]
    f_hid = W1.shape[1]
    f_out = W2.shape[1]

    n_acc = ((n + 1 + _NS * 8 - 1) // (_NS * 8)) * (_NS * 8)
    while n_acc % 128 != 0:
        n_acc += _NS * 8

    per = _NC * _NS * 2 * _G * _CHUNK
    n_chunks = ((e + per - 1) // per) * per // _CHUNK
    cpw = n_chunks // (_NC * _NS)
    n_grps = cpw // _G
    ep = n_chunks * _CHUNK

    pad = jnp.full((ep - e,), n, dtype=jnp.int32)
    src2d = jnp.concatenate([edge_index[0], pad]).reshape(n_chunks, _CHUNK)
    dst2d = jnp.concatenate([edge_index[1], pad]).reshape(n_chunks, _CHUNK)

    xp = jnp.zeros((n_acc, _D), jnp.float32).at[:n, :f_in].set(x)
    w1p = jnp.zeros((_D, f_hid), jnp.float32).at[:f_in].set(W1)
    w2p = jnp.zeros((f_hid, _D), jnp.float32).at[:, :f_out].set(W2)
    b2p = jnp.zeros((1, _D), jnp.float32).at[0, :f_out].set(b2)
    zeros8 = jnp.zeros((n_acc, _D), jnp.float32)
    ones = jnp.ones((_CHUNK, _D), jnp.float32)

    degp = _deg_pass(n_acc, cpw, n_grps)(dst2d, zeros8, ones)

    rblk = n_acc // 32 if n_acc % 32 == 0 else n_acc
    while n_acc % rblk or rblk % 8:
        rblk //= 2
    grid = n_acc // rblk
    dinv, u1 = pl.pallas_call(
        _tc1_body,
        grid=(grid,),
        in_specs=_row_specs(rblk, [(_NC, n_acc, _D), (None, _D)]),
        out_specs=_row_specs(rblk, [(None, 1), (None, _D)]),
        out_shape=[jax.ShapeDtypeStruct((n_acc, 1), jnp.float32),
                   jax.ShapeDtypeStruct((n_acc, _D), jnp.float32)],
    )(degp, xp)

    acc1 = _agg_pass(n_acc, cpw, n_grps)(src2d, dst2d, u1, zeros8)

    u2 = pl.pallas_call(
        _tc2_body,
        grid=(grid,),
        in_specs=_row_specs(rblk, [
            (_NC, n_acc, _D), (None, _D), (None, 1),
            ("full", (_D, f_hid)), ("full", (1, f_hid)),
            ("full", (f_hid, _D))]),
        out_specs=_row_specs(rblk, [(None, _D)]),
        out_shape=[jax.ShapeDtypeStruct((n_acc, _D), jnp.float32)],
    )(acc1, u1, dinv, w1p, b1.reshape(1, f_hid), w2p)[0]

    acc2 = _agg_pass(n_acc, cpw, n_grps)(src2d, dst2d, u2, zeros8)

    out = pl.pallas_call(
        _tc3_body,
        grid=(grid,),
        in_specs=_row_specs(rblk, [
            (_NC, n_acc, _D), (None, _D), (None, 1),
            ("full", (1, _D))]),
        out_specs=_row_specs(rblk, [(None, _D)]),
        out_shape=[jax.ShapeDtypeStruct((n_acc, _D), jnp.float32)],
    )(acc2, u2, dinv, b2p)[0]

    return out[:n, :f_out]


# speedup vs baseline: 1.6432x; 1.2576x over previous
import functools

import jax
import jax.numpy as jnp
from jax import lax
from jax.experimental import pallas as pl
from jax.experimental.pallas import tpu as pltpu
from jax.experimental.pallas import tpu_sc as plsc

_NC = 2
_NS = 16
_CHUNK = 128
_G = 16
_GA = 8
_D = 8


def _mesh():
    return plsc.VectorSubcoreMesh(core_axis_name="c", subcore_axis_name="s")


_SC_PARAMS = pltpu.CompilerParams(use_tc_tiling_on_sc=False)


def _deg_pass(n_acc, cpw, n_grps):
    stripe = n_acc // _NS

    @functools.partial(
        pl.kernel,
        out_type=jax.ShapeDtypeStruct((_NC, n_acc, _D), jnp.float32),
        mesh=_mesh(),
        compiler_params=_SC_PARAMS,
        scratch_types=[
            pltpu.VMEM_SHARED((n_acc, _D), jnp.float32),
            pltpu.VMEM((_CHUNK, _D), jnp.float32),
            pltpu.VMEM((_G, _CHUNK), jnp.int32),
            pltpu.VMEM((_G, _CHUNK), jnp.int32),
            pltpu.SemaphoreType.DMA,
            pltpu.SemaphoreType.DMA,
        ],
    )
    def k(dst2d, zeros_hbm, ones_hbm, out, acc_sh, onesv, db0, db1, ss0, ss1):
        dbs, sss = (db0, db1), (ss0, ss1)
        c = lax.axis_index("c")
        s = lax.axis_index("s")
        r0 = s * stripe
        pltpu.sync_copy(zeros_hbm.at[pl.ds(r0, stripe)],
                        acc_sh.at[pl.ds(r0, stripe)])
        pltpu.sync_copy(ones_hbm, onesv)
        plsc.subcore_barrier()
        e0 = (c * _NS + s) * cpw
        dummy = zeros_hbm.at[pl.ds(0, _CHUNK)]

        def grp2(i, carry):
            for p in range(2):
                dbuf, ssem = dbs[p], sss[p]

                @pl.when(i > 0)
                def _():
                    for g in range(_G):
                        pltpu.make_async_copy(dummy, onesv, ssem).wait()

                base = e0 + (2 * i + p) * _G
                pltpu.sync_copy(dst2d.at[pl.ds(base, _G)], dbuf)
                for g in range(_G):
                    pltpu.async_copy(onesv, acc_sh.at[dbuf.at[g]], ssem,
                                     add=True)
            return carry

        lax.fori_loop(0, n_grps // 2, grp2, 0)
        for p in range(2):
            for g in range(_G):
                pltpu.make_async_copy(dummy, onesv, sss[p]).wait()
        plsc.subcore_barrier()
        pltpu.sync_copy(acc_sh.at[pl.ds(r0, stripe)],
                        out.at[c, pl.ds(r0, stripe)])

    return k


def _agg_pass(n_acc, cpw, n_grps):
    stripe = n_acc // _NS

    @functools.partial(
        pl.kernel,
        out_type=jax.ShapeDtypeStruct((_NC, n_acc, _D), jnp.float32),
        mesh=_mesh(),
        compiler_params=_SC_PARAMS,
        scratch_types=[
            pltpu.VMEM_SHARED((n_acc, _D), jnp.float32),
            pltpu.VMEM_SHARED((n_acc, _D), jnp.float32),
            pltpu.VMEM((_GA, _CHUNK, _D), jnp.float32),
            pltpu.VMEM((_GA, _CHUNK, _D), jnp.float32),
            pltpu.VMEM((_GA, _CHUNK), jnp.int32),
            pltpu.VMEM((_GA, _CHUNK), jnp.int32),
            pltpu.VMEM((_GA, _CHUNK), jnp.int32),
            pltpu.VMEM((_GA, _CHUNK), jnp.int32),
            pltpu.SemaphoreType.DMA,
            pltpu.SemaphoreType.DMA,
            pltpu.SemaphoreType.DMA,
        ],
    )
    def k(src2d, dst2d, table, zeros_hbm, out, acc_sh, tab_sh,
          rows0, rows1, sb0, sb1, db0, db1, gsem, ss0, ss1):
        rws, sbs, dbs, sss = (rows0, rows1), (sb0, sb1), (db0, db1), (ss0, ss1)
        c = lax.axis_index("c")
        s = lax.axis_index("s")
        r0 = s * stripe
        pltpu.sync_copy(zeros_hbm.at[pl.ds(r0, stripe)],
                        acc_sh.at[pl.ds(r0, stripe)])
        pltpu.sync_copy(table.at[pl.ds(r0, stripe)],
                        tab_sh.at[pl.ds(r0, stripe)])
        plsc.subcore_barrier()
        e0 = (c * _NS + s) * cpw
        dummy = zeros_hbm.at[pl.ds(0, _CHUNK)]

        def grp2(i, carry):
            for p in range(2):
                rows, sbuf, dbuf, ssem = rws[p], sbs[p], dbs[p], sss[p]

                @pl.when(i > 0)
                def _():
                    for g in range(_GA):
                        pltpu.make_async_copy(dummy, rows.at[g], ssem).wait()

                base = e0 + (2 * i + p) * _GA
                pltpu.sync_copy(src2d.at[pl.ds(base, _GA)], sbuf)
                pltpu.sync_copy(dst2d.at[pl.ds(base, _GA)], dbuf)
                descs = [pltpu.async_copy(tab_sh.at[sbuf.at[g]], rows.at[g],
                                          gsem) for g in range(_GA)]
                for d in descs:
                    d.wait()
                for g in range(_GA):
                    pltpu.async_copy(rows.at[g], acc_sh.at[dbuf.at[g]], ssem,
                                     add=True)
            return carry

        lax.fori_loop(0, n_grps // 2, grp2, 0)
        for p in range(2):
            for g in range(_GA):
                pltpu.make_async_copy(dummy, rws[p].at[g], sss[p]).wait()
        plsc.subcore_barrier()
        pltpu.sync_copy(acc_sh.at[pl.ds(r0, stripe)],
                        out.at[c, pl.ds(r0, stripe)])

    return k


def _tc1_body(degp_ref, x_ref, dinv_ref, u1_ref):
    deg = degp_ref[0, :, 0:1] + degp_ref[1, :, 0:1] + 1.0
    dinv = lax.rsqrt(deg)
    dinv_ref[...] = dinv
    u1_ref[...] = x_ref[...] * dinv


def _tc2_body(accp_ref, u1_ref, dinv_ref, w1_ref, b1_ref, w2_ref, u2_ref):
    dinv = dinv_ref[...]
    a1 = dinv * (accp_ref[0] + accp_ref[1] + u1_ref[...])
    h = jnp.maximum(
        jnp.dot(a1, w1_ref[...], preferred_element_type=jnp.float32)
        + b1_ref[...], 0.0)
    p = jnp.dot(h, w2_ref[...], preferred_element_type=jnp.float32)
    u2_ref[...] = p * dinv


def _tc3_body(accp_ref, u2_ref, dinv_ref, b2_ref, out_ref):
    out_ref[...] = (dinv_ref[...] * (accp_ref[0] + accp_ref[1] + u2_ref[...])
                    + b2_ref[...])


def _row_specs(rblk, shapes):
    specs = []
    for shp in shapes:
        if len(shp) == 3:
            specs.append(pl.BlockSpec((_NC, rblk, shp[2]),
                                      lambda i: (0, i, 0)))
        elif shp[0] == "full":
            specs.append(pl.BlockSpec(shp[1], lambda i, s=shp[1]: (0,) * len(s)))
        else:
            specs.append(pl.BlockSpec((rblk, shp[1]), lambda i: (i, 0)))
    return specs


def kernel(x, edge_index, W1, b1, W2, b2):
    n = x.shape[0]
    e = edge_index.shape[1]
    f_in = x.shape[1]
    f_hid = W1.shape[1]
    f_out = W2.shape[1]

    n_acc = ((n + 1 + _NS * 8 - 1) // (_NS * 8)) * (_NS * 8)
    while n_acc % 128 != 0:
        n_acc += _NS * 8

    per = _NC * _NS * 2 * _G * _CHUNK
    n_chunks = ((e + per - 1) // per) * per // _CHUNK
    cpw = n_chunks // (_NC * _NS)
    n_grps = cpw // _G
    n_grps_a = cpw // _GA
    ep = n_chunks * _CHUNK

    pad = jnp.full((ep - e,), n, dtype=jnp.int32)
    src2d = jnp.concatenate([edge_index[0], pad]).reshape(n_chunks, _CHUNK)
    dst2d = jnp.concatenate([edge_index[1], pad]).reshape(n_chunks, _CHUNK)

    xp = jnp.zeros((n_acc, _D), jnp.float32).at[:n, :f_in].set(x)
    w1p = jnp.zeros((_D, f_hid), jnp.float32).at[:f_in].set(W1)
    w2p = jnp.zeros((f_hid, _D), jnp.float32).at[:, :f_out].set(W2)
    b2p = jnp.zeros((1, _D), jnp.float32).at[0, :f_out].set(b2)
    zeros8 = jnp.zeros((n_acc, _D), jnp.float32)
    ones = jnp.ones((_CHUNK, _D), jnp.float32)

    degp = _deg_pass(n_acc, cpw, n_grps)(dst2d, zeros8, ones)

    rblk = n_acc // 32 if n_acc % 32 == 0 else n_acc
    while n_acc % rblk or rblk % 8:
        rblk //= 2
    grid = n_acc // rblk
    dinv, u1 = pl.pallas_call(
        _tc1_body,
        grid=(grid,),
        in_specs=_row_specs(rblk, [(_NC, n_acc, _D), (None, _D)]),
        out_specs=_row_specs(rblk, [(None, 1), (None, _D)]),
        out_shape=[jax.ShapeDtypeStruct((n_acc, 1), jnp.float32),
                   jax.ShapeDtypeStruct((n_acc, _D), jnp.float32)],
    )(degp, xp)

    acc1 = _agg_pass(n_acc, cpw, n_grps_a)(src2d, dst2d, u1, zeros8)

    u2 = pl.pallas_call(
        _tc2_body,
        grid=(grid,),
        in_specs=_row_specs(rblk, [
            (_NC, n_acc, _D), (None, _D), (None, 1),
            ("full", (_D, f_hid)), ("full", (1, f_hid)),
            ("full", (f_hid, _D))]),
        out_specs=_row_specs(rblk, [(None, _D)]),
        out_shape=[jax.ShapeDtypeStruct((n_acc, _D), jnp.float32)],
    )(acc1, u1, dinv, w1p, b1.reshape(1, f_hid), w2p)[0]

    acc2 = _agg_pass(n_acc, cpw, n_grps_a)(src2d, dst2d, u2, zeros8)

    out = pl.pallas_call(
        _tc3_body,
        grid=(grid,),
        in_specs=_row_specs(rblk, [
            (_NC, n_acc, _D), (None, _D), (None, 1),
            ("full", (1, _D))]),
        out_specs=_row_specs(rblk, [(None, _D)]),
        out_shape=[jax.ShapeDtypeStruct((n_acc, _D), jnp.float32)],
    )(acc2, u2, dinv, b2p)[0]

    return out[:n, :f_out]
